# baseline (speedup 1.0000x reference)
import jax
import jax.numpy as jnp
from jax.experimental import pallas as pl
from jax.experimental.pallas import tpu as pltpu


def _gating_body(x_ref, wg_ref, be_ref, gates_ref, loss_ref, bc_ref, gx_ref):
    i = pl.program_id(0)
    n = pl.num_programs(0)
    gb = x_ref.shape[0]
    base = pl.multiple_of(i * gb, 8)
    gx_ref[pl.ds(base, gb), :] = jnp.mean(x_ref[...], axis=2)

    @pl.when(i == n - 1)
    def _():
        B = gx_ref.shape[0]
        E = wg_ref.shape[1]
        logits = jnp.dot(gx_ref[...], wg_ref[...],
                         preferred_element_type=jnp.float32)
        z = jnp.exp(logits - jnp.max(logits, axis=1, keepdims=True))
        sm = z / jnp.sum(z, axis=1, keepdims=True)
        iota = jax.lax.broadcasted_iota(jnp.int32, (B, E), 1)
        m1 = jnp.max(sm, axis=1, keepdims=True)
        i1 = jnp.min(jnp.where(sm == m1, iota, E), axis=1, keepdims=True)
        masked = jnp.where(iota == i1, -1.0, sm)
        m2 = jnp.max(masked, axis=1, keepdims=True)
        i2 = jnp.min(jnp.where(masked == m2, iota, E), axis=1, keepdims=True)
        denom = m1 + m2 + 1e-6
        gates = (jnp.where(iota == i1, m1 / denom, 0.0)
                 + jnp.where(iota == i2, m2 / denom, 0.0))
        gates_ref[...] = gates
        bc_ref[...] = jnp.dot(gates, be_ref[...],
                              preferred_element_type=jnp.float32)

        def cv_sq(v):
            mu = jnp.mean(v, keepdims=True)
            var = jnp.sum((v - mu) ** 2, keepdims=True) / (E - 1)
            return var / (mu ** 2 + 1e-10)

        imp = jnp.sum(gates, axis=0, keepdims=True)
        load = jnp.sum((gates > 0.0).astype(jnp.float32), axis=0,
                       keepdims=True)
        loss_ref[...] = (cv_sq(imp) + cv_sq(load)) * 0.01


def _combine_body(g_ref, w2_ref, wc_ref):
    wc_ref[...] = jnp.dot(g_ref[...], w2_ref[...],
                          preferred_element_type=jnp.float32)


def _make_conv_body(C_IN, C_OUT, H, W, KS):
    HW = H * W

    def conv_body(x_ref, wc_ref, bc_ref, out_ref):
        col = jax.lax.broadcasted_iota(jnp.int32, (1, HW), 1) % W
        mask_l = (col != 0).astype(jnp.float32)
        mask_r = (col != W - 1).astype(jnp.float32)

        p_all = jnp.dot(wc_ref[0], x_ref[0],
                        preferred_element_type=jnp.float32)
        acc = None
        for t in range(KS * KS):
            kh, kw = t // KS, t % KS
            d = W * (kh - 1) + (kw - 1)
            p = p_all[t * C_OUT:(t + 1) * C_OUT, :]
            if d > 0:
                p = jnp.concatenate(
                    [p[:, d:], jnp.zeros((C_OUT, d), jnp.float32)], axis=1)
            elif d < 0:
                p = jnp.concatenate(
                    [jnp.zeros((C_OUT, -d), jnp.float32), p[:, :HW + d]],
                    axis=1)
            if kw == 0:
                p = p * mask_l
            elif kw == KS - 1:
                p = p * mask_r
            acc = p if acc is None else acc + p
        out_ref[0] = acc + bc_ref[0]

    return conv_body


def kernel(x, w_gate, W_exp, b_exp):
    B, C_IN, H, W = x.shape
    E = w_gate.shape[1]
    C_OUT = W_exp.shape[1]
    KS = W_exp.shape[3]
    HW = H * W
    f32 = jnp.float32

    GB = 8
    x3 = x.reshape(B, C_IN, HW)
    gates, loss_arr, b_c = pl.pallas_call(
        _gating_body,
        grid=(B // GB,),
        in_specs=[
            pl.BlockSpec((GB, C_IN, HW), lambda i: (i, 0, 0)),
            pl.BlockSpec((C_IN, E), lambda i: (0, 0)),
            pl.BlockSpec((E, C_OUT), lambda i: (0, 0)),
        ],
        out_specs=[
            pl.BlockSpec((B, E), lambda i: (0, 0)),
            pl.BlockSpec((1, 1), lambda i: (0, 0)),
            pl.BlockSpec((B, C_OUT), lambda i: (0, 0)),
        ],
        out_shape=[
            jax.ShapeDtypeStruct((B, E), f32),
            jax.ShapeDtypeStruct((1, 1), f32),
            jax.ShapeDtypeStruct((B, C_OUT), f32),
        ],
        scratch_shapes=[pltpu.VMEM((B, C_IN), f32)],
    )(x3, w_gate, b_exp)

    W2 = W_exp.transpose(0, 3, 4, 1, 2).reshape(E, KS * KS * C_OUT * C_IN)
    NCH = 8
    CHUNK = W2.shape[1] // NCH
    W_c = pl.pallas_call(
        _combine_body,
        grid=(NCH,),
        in_specs=[
            pl.BlockSpec((B, E), lambda j: (0, 0)),
            pl.BlockSpec((E, CHUNK), lambda j: (0, j)),
        ],
        out_specs=pl.BlockSpec((B, CHUNK), lambda j: (0, j)),
        out_shape=jax.ShapeDtypeStruct((B, KS * KS * C_OUT * C_IN), f32),
    )(gates, W2)

    Wc3 = W_c.reshape(B, KS * KS * C_OUT, C_IN)
    bc3 = b_c.reshape(B, C_OUT, 1)
    y_flat = pl.pallas_call(
        _make_conv_body(C_IN, C_OUT, H, W, KS),
        grid=(B,),
        in_specs=[
            pl.BlockSpec((1, C_IN, HW), lambda b: (b, 0, 0)),
            pl.BlockSpec((1, KS * KS * C_OUT, C_IN), lambda b: (b, 0, 0)),
            pl.BlockSpec((1, C_OUT, 1), lambda b: (b, 0, 0)),
        ],
        out_specs=pl.BlockSpec((1, C_OUT, HW), lambda b: (b, 0, 0)),
        out_shape=jax.ShapeDtypeStruct((B, C_OUT, HW), f32),
    )(x3, Wc3, bc3)
    y = y_flat.reshape(B, C_OUT, H, W)

    return (y, loss_arr[0, 0])


# speedup vs baseline: 1.1654x; 1.1654x over previous
import jax
import jax.numpy as jnp
from jax.experimental import pallas as pl
from jax.experimental.pallas import tpu as pltpu


def _gating_body(x_ref, wg_ref, be_ref, idx_ref, gg_ref, loss_ref, bc_ref,
                 gx_ref):
    i = pl.program_id(0)
    n = pl.num_programs(0)
    gb = x_ref.shape[0]
    base = pl.multiple_of(i * gb, 8)
    gx_ref[pl.ds(base, gb), :] = jnp.mean(x_ref[...], axis=2)

    @pl.when(i == n - 1)
    def _():
        B = gx_ref.shape[0]
        E = wg_ref.shape[1]
        logits = jnp.dot(gx_ref[...], wg_ref[...],
                         preferred_element_type=jnp.float32)
        z = jnp.exp(logits - jnp.max(logits, axis=1, keepdims=True))
        sm = z / jnp.sum(z, axis=1, keepdims=True)
        iota = jax.lax.broadcasted_iota(jnp.int32, (B, E), 1)
        m1 = jnp.max(sm, axis=1, keepdims=True)
        i1 = jnp.min(jnp.where(sm == m1, iota, E), axis=1, keepdims=True)
        masked = jnp.where(iota == i1, -1.0, sm)
        m2 = jnp.max(masked, axis=1, keepdims=True)
        i2 = jnp.min(jnp.where(masked == m2, iota, E), axis=1, keepdims=True)
        denom = m1 + m2 + 1e-6
        g1 = m1 / denom
        g2 = m2 / denom
        gates = (jnp.where(iota == i1, g1, 0.0)
                 + jnp.where(iota == i2, g2, 0.0))
        idx_ref[...] = jnp.concatenate([i1, i2], axis=1)
        gg_ref[...] = jnp.concatenate([g1, g2], axis=1)
        bc_ref[...] = jnp.dot(gates, be_ref[...],
                              preferred_element_type=jnp.float32)

        def cv_sq(v):
            mu = jnp.mean(v, keepdims=True)
            var = jnp.sum((v - mu) ** 2, keepdims=True) / (E - 1)
            return var / (mu ** 2 + 1e-10)

        imp = jnp.sum(gates, axis=0, keepdims=True)
        load = jnp.sum((gates > 0.0).astype(jnp.float32), axis=0,
                       keepdims=True)
        loss_ref[...] = (cv_sq(imp) + cv_sq(load)) * 0.01


def _make_conv_body(C_IN, C_OUT, H, W, KS):
    HW = H * W

    def conv_body(idx_ref, gg_ref, x_ref, w1_ref, w2_ref, bc_ref, out_ref):
        b = pl.program_id(0)
        col = jax.lax.broadcasted_iota(jnp.int32, (1, HW), 1) % W
        mask_l = (col != 0).astype(jnp.float32)
        mask_r = (col != W - 1).astype(jnp.float32)

        wc = gg_ref[b, 0] * w1_ref[0] + gg_ref[b, 1] * w2_ref[0]
        p_all = jnp.dot(wc, x_ref[0], preferred_element_type=jnp.float32)
        acc = None
        for t in range(KS * KS):
            kh, kw = t // KS, t % KS
            d = W * (kh - 1) + (kw - 1)
            p = p_all[t * C_OUT:(t + 1) * C_OUT, :]
            if d > 0:
                p = jnp.concatenate(
                    [p[:, d:], jnp.zeros((C_OUT, d), jnp.float32)], axis=1)
            elif d < 0:
                p = jnp.concatenate(
                    [jnp.zeros((C_OUT, -d), jnp.float32), p[:, :HW + d]],
                    axis=1)
            if kw == 0:
                p = p * mask_l
            elif kw == KS - 1:
                p = p * mask_r
            acc = p if acc is None else acc + p
        out_ref[0] = acc + bc_ref[0]

    return conv_body


def kernel(x, w_gate, W_exp, b_exp):
    B, C_IN, H, W = x.shape
    E = w_gate.shape[1]
    C_OUT = W_exp.shape[1]
    KS = W_exp.shape[3]
    HW = H * W
    f32 = jnp.float32

    GB = 8
    x3 = x.reshape(B, C_IN, HW)
    idx2, gg2, loss_arr, b_c = pl.pallas_call(
        _gating_body,
        grid=(B // GB,),
        in_specs=[
            pl.BlockSpec((GB, C_IN, HW), lambda i: (i, 0, 0)),
            pl.BlockSpec((C_IN, E), lambda i: (0, 0)),
            pl.BlockSpec((E, C_OUT), lambda i: (0, 0)),
        ],
        out_specs=[
            pl.BlockSpec((B, 2), lambda i: (0, 0)),
            pl.BlockSpec((B, 2), lambda i: (0, 0)),
            pl.BlockSpec((1, 1), lambda i: (0, 0)),
            pl.BlockSpec((B, C_OUT), lambda i: (0, 0)),
        ],
        out_shape=[
            jax.ShapeDtypeStruct((B, 2), jnp.int32),
            jax.ShapeDtypeStruct((B, 2), f32),
            jax.ShapeDtypeStruct((1, 1), f32),
            jax.ShapeDtypeStruct((B, C_OUT), f32),
        ],
        scratch_shapes=[pltpu.VMEM((B, C_IN), f32)],
    )(x3, w_gate, b_exp)

    W2 = W_exp.transpose(0, 3, 4, 1, 2).reshape(E, KS * KS * C_OUT, C_IN)
    bc3 = b_c.reshape(B, C_OUT, 1)
    grid_spec = pltpu.PrefetchScalarGridSpec(
        num_scalar_prefetch=2,
        grid=(B,),
        in_specs=[
            pl.BlockSpec((1, C_IN, HW), lambda b, idx, gg: (b, 0, 0)),
            pl.BlockSpec((1, KS * KS * C_OUT, C_IN),
                         lambda b, idx, gg: (idx[b, 0], 0, 0)),
            pl.BlockSpec((1, KS * KS * C_OUT, C_IN),
                         lambda b, idx, gg: (idx[b, 1], 0, 0)),
            pl.BlockSpec((1, C_OUT, 1), lambda b, idx, gg: (b, 0, 0)),
        ],
        out_specs=pl.BlockSpec((1, C_OUT, HW), lambda b, idx, gg: (b, 0, 0)),
    )
    y_flat = pl.pallas_call(
        _make_conv_body(C_IN, C_OUT, H, W, KS),
        grid_spec=grid_spec,
        out_shape=jax.ShapeDtypeStruct((B, C_OUT, HW), f32),
    )(idx2, gg2, x3, W2, W2, bc3)
    y = y_flat.reshape(B, C_OUT, H, W)

    return (y, loss_arr[0, 0])


# speedup vs baseline: 1.3446x; 1.1538x over previous
import jax
import jax.numpy as jnp
from jax.experimental import pallas as pl
from jax.experimental.pallas import tpu as pltpu


def _gating_body(x_ref, wg_ref, be_ref, idx_ref, gg_ref, loss_ref, bc_ref,
                 gx_ref):
    i = pl.program_id(0)
    n = pl.num_programs(0)
    gb = x_ref.shape[0]
    base = pl.multiple_of(i * gb, 8)
    gx_ref[pl.ds(base, gb), :] = jnp.mean(x_ref[...], axis=2)

    @pl.when(i == n - 1)
    def _():
        B = gx_ref.shape[0]
        E = wg_ref.shape[1]
        logits = jnp.dot(gx_ref[...], wg_ref[...],
                         preferred_element_type=jnp.float32)
        z = jnp.exp(logits - jnp.max(logits, axis=1, keepdims=True))
        sm = z / jnp.sum(z, axis=1, keepdims=True)
        iota = jax.lax.broadcasted_iota(jnp.int32, (B, E), 1)
        m1 = jnp.max(sm, axis=1, keepdims=True)
        i1 = jnp.min(jnp.where(sm == m1, iota, E), axis=1, keepdims=True)
        masked = jnp.where(iota == i1, -1.0, sm)
        m2 = jnp.max(masked, axis=1, keepdims=True)
        i2 = jnp.min(jnp.where(masked == m2, iota, E), axis=1, keepdims=True)
        denom = m1 + m2 + 1e-6
        g1 = m1 / denom
        g2 = m2 / denom
        gates = (jnp.where(iota == i1, g1, 0.0)
                 + jnp.where(iota == i2, g2, 0.0))
        idx_ref[...] = jnp.concatenate([i1, i2], axis=1)
        gg_ref[...] = jnp.concatenate([g1, g2], axis=1)
        bc_ref[...] = jnp.dot(gates, be_ref[...],
                              preferred_element_type=jnp.float32)

        def cv_sq(v):
            mu = jnp.mean(v, keepdims=True)
            var = jnp.sum((v - mu) ** 2, keepdims=True) / (E - 1)
            return var / (mu ** 2 + 1e-10)

        imp = jnp.sum(gates, axis=0, keepdims=True)
        load = jnp.sum((gates > 0.0).astype(jnp.float32), axis=0,
                       keepdims=True)
        loss_ref[...] = (cv_sq(imp) + cv_sq(load)) * 0.01


def _make_conv_body(C_IN, C_OUT, H, W, KS):
    HW = H * W

    def conv_body(idx_ref, gg_ref, x_ref, w1a_ref, w2a_ref, w1b_ref,
                  w2b_ref, bc_ref, out_ref):
        s = pl.program_id(0)
        col = jax.lax.broadcasted_iota(jnp.int32, (1, HW), 1) % W
        mask_l = (col != 0).astype(jnp.float32)
        mask_r = (col != W - 1).astype(jnp.float32)

        for j, (wp_ref, wq_ref) in enumerate(
                [(w1a_ref, w2a_ref), (w1b_ref, w2b_ref)]):
            b = 2 * s + j
            wc = gg_ref[b, 0] * wp_ref[0] + gg_ref[b, 1] * wq_ref[0]
            p_all = jnp.dot(wc, x_ref[j],
                            preferred_element_type=jnp.float32)
            acc = None
            for t in range(KS * KS):
                kh, kw = t // KS, t % KS
                d = W * (kh - 1) + (kw - 1)
                p = p_all[t * C_OUT:(t + 1) * C_OUT, :]
                if d > 0:
                    p = jnp.concatenate(
                        [p[:, d:], jnp.zeros((C_OUT, d), jnp.float32)],
                        axis=1)
                elif d < 0:
                    p = jnp.concatenate(
                        [jnp.zeros((C_OUT, -d), jnp.float32),
                         p[:, :HW + d]], axis=1)
                if kw == 0:
                    p = p * mask_l
                elif kw == KS - 1:
                    p = p * mask_r
                acc = p if acc is None else acc + p
            out_ref[j] = acc + bc_ref[j]

    return conv_body


def kernel(x, w_gate, W_exp, b_exp):
    B, C_IN, H, W = x.shape
    E = w_gate.shape[1]
    C_OUT = W_exp.shape[1]
    KS = W_exp.shape[3]
    HW = H * W
    f32 = jnp.float32

    GB = 16
    x3 = x.reshape(B, C_IN, HW)
    idx2, gg2, loss_arr, b_c = pl.pallas_call(
        _gating_body,
        grid=(B // GB,),
        in_specs=[
            pl.BlockSpec((GB, C_IN, HW), lambda i: (i, 0, 0)),
            pl.BlockSpec((C_IN, E), lambda i: (0, 0)),
            pl.BlockSpec((E, C_OUT), lambda i: (0, 0)),
        ],
        out_specs=[
            pl.BlockSpec((B, 2), lambda i: (0, 0)),
            pl.BlockSpec((B, 2), lambda i: (0, 0)),
            pl.BlockSpec((1, 1), lambda i: (0, 0)),
            pl.BlockSpec((B, C_OUT), lambda i: (0, 0)),
        ],
        out_shape=[
            jax.ShapeDtypeStruct((B, 2), jnp.int32),
            jax.ShapeDtypeStruct((B, 2), f32),
            jax.ShapeDtypeStruct((1, 1), f32),
            jax.ShapeDtypeStruct((B, C_OUT), f32),
        ],
        scratch_shapes=[pltpu.VMEM((B, C_IN), f32)],
    )(x3, w_gate, b_exp)

    W2 = W_exp.transpose(0, 3, 4, 1, 2).reshape(E, KS * KS * C_OUT, C_IN)
    bc3 = b_c.reshape(B, C_OUT, 1)
    WB = KS * KS * C_OUT
    grid_spec = pltpu.PrefetchScalarGridSpec(
        num_scalar_prefetch=2,
        grid=(B // 2,),
        in_specs=[
            pl.BlockSpec((2, C_IN, HW), lambda s, idx, gg: (s, 0, 0)),
            pl.BlockSpec((1, WB, C_IN),
                         lambda s, idx, gg: (idx[2 * s, 0], 0, 0)),
            pl.BlockSpec((1, WB, C_IN),
                         lambda s, idx, gg: (idx[2 * s, 1], 0, 0)),
            pl.BlockSpec((1, WB, C_IN),
                         lambda s, idx, gg: (idx[2 * s + 1, 0], 0, 0)),
            pl.BlockSpec((1, WB, C_IN),
                         lambda s, idx, gg: (idx[2 * s + 1, 1], 0, 0)),
            pl.BlockSpec((2, C_OUT, 1), lambda s, idx, gg: (s, 0, 0)),
        ],
        out_specs=pl.BlockSpec((2, C_OUT, HW), lambda s, idx, gg: (s, 0, 0)),
    )
    y_flat = pl.pallas_call(
        _make_conv_body(C_IN, C_OUT, H, W, KS),
        grid_spec=grid_spec,
        out_shape=jax.ShapeDtypeStruct((B, C_OUT, HW), f32),
    )(idx2, gg2, x3, W2, W2, W2, W2, bc3)
    y = y_flat.reshape(B, C_OUT, H, W)

    return (y, loss_arr[0, 0])


# speedup vs baseline: 1.3823x; 1.0280x over previous
import jax
import jax.numpy as jnp
from jax.experimental import pallas as pl
from jax.experimental.pallas import tpu as pltpu


def _gating_body(x_ref, wg_ref, be_ref, idx_ref, gg_ref, loss_ref, bc_ref,
                 gx_ref):
    i = pl.program_id(0)
    n = pl.num_programs(0)
    gb = x_ref.shape[0]
    base = pl.multiple_of(i * gb, 8)
    gx_ref[pl.ds(base, gb), :] = jnp.mean(x_ref[...], axis=2)

    @pl.when(i == n - 1)
    def _():
        B = gx_ref.shape[0]
        E = wg_ref.shape[1]
        logits = jnp.dot(gx_ref[...], wg_ref[...],
                         preferred_element_type=jnp.float32)
        z = jnp.exp(logits - jnp.max(logits, axis=1, keepdims=True))
        sm = z / jnp.sum(z, axis=1, keepdims=True)
        iota = jax.lax.broadcasted_iota(jnp.int32, (B, E), 1)
        m1 = jnp.max(sm, axis=1, keepdims=True)
        i1 = jnp.min(jnp.where(sm == m1, iota, E), axis=1, keepdims=True)
        masked = jnp.where(iota == i1, -1.0, sm)
        m2 = jnp.max(masked, axis=1, keepdims=True)
        i2 = jnp.min(jnp.where(masked == m2, iota, E), axis=1, keepdims=True)
        denom = m1 + m2 + 1e-6
        g1 = m1 / denom
        g2 = m2 / denom
        gates = (jnp.where(iota == i1, g1, 0.0)
                 + jnp.where(iota == i2, g2, 0.0))
        idx_ref[...] = jnp.concatenate([i1, i2], axis=1)
        gg_ref[...] = jnp.concatenate([g1, g2], axis=1)
        bc_ref[...] = jnp.dot(gates, be_ref[...],
                              preferred_element_type=jnp.float32)

        def cv_sq(v):
            mu = jnp.mean(v, keepdims=True)
            var = jnp.sum((v - mu) ** 2, keepdims=True) / (E - 1)
            return var / (mu ** 2 + 1e-10)

        imp = jnp.sum(gates, axis=0, keepdims=True)
        load = jnp.sum((gates > 0.0).astype(jnp.float32), axis=0,
                       keepdims=True)
        loss_ref[...] = (cv_sq(imp) + cv_sq(load)) * 0.01


def _make_conv_body(C_IN, C_OUT, H, W, KS):
    HW = H * W

    def conv_body(idx_ref, gg_ref, x_ref, *w_bc_out_refs):
        BB = x_ref.shape[0]
        w_refs = w_bc_out_refs[:2 * BB]
        bc_ref, out_ref = w_bc_out_refs[2 * BB:]
        s = pl.program_id(0)
        col = jax.lax.broadcasted_iota(jnp.int32, (1, HW), 1) % W
        mask_l = (col != 0).astype(jnp.float32)
        mask_r = (col != W - 1).astype(jnp.float32)

        for j in range(BB):
            wp_ref, wq_ref = w_refs[2 * j], w_refs[2 * j + 1]
            b = BB * s + j
            wc = gg_ref[b, 0] * wp_ref[0] + gg_ref[b, 1] * wq_ref[0]
            p_all = jnp.dot(wc, x_ref[j],
                            preferred_element_type=jnp.float32)
            acc = None
            for t in range(KS * KS):
                kh, kw = t // KS, t % KS
                d = W * (kh - 1) + (kw - 1)
                p = p_all[t * C_OUT:(t + 1) * C_OUT, :]
                if d > 0:
                    p = jnp.concatenate(
                        [p[:, d:], jnp.zeros((C_OUT, d), jnp.float32)],
                        axis=1)
                elif d < 0:
                    p = jnp.concatenate(
                        [jnp.zeros((C_OUT, -d), jnp.float32),
                         p[:, :HW + d]], axis=1)
                if kw == 0:
                    p = p * mask_l
                elif kw == KS - 1:
                    p = p * mask_r
                acc = p if acc is None else acc + p
            out_ref[j] = acc + bc_ref[j]

    return conv_body


def kernel(x, w_gate, W_exp, b_exp):
    B, C_IN, H, W = x.shape
    E = w_gate.shape[1]
    C_OUT = W_exp.shape[1]
    KS = W_exp.shape[3]
    HW = H * W
    f32 = jnp.float32

    GB = 16
    x3 = x.reshape(B, C_IN, HW)
    idx2, gg2, loss_arr, b_c = pl.pallas_call(
        _gating_body,
        grid=(B // GB,),
        in_specs=[
            pl.BlockSpec((GB, C_IN, HW), lambda i: (i, 0, 0)),
            pl.BlockSpec((C_IN, E), lambda i: (0, 0)),
            pl.BlockSpec((E, C_OUT), lambda i: (0, 0)),
        ],
        out_specs=[
            pl.BlockSpec((B, 2), lambda i: (0, 0)),
            pl.BlockSpec((B, 2), lambda i: (0, 0)),
            pl.BlockSpec((1, 1), lambda i: (0, 0)),
            pl.BlockSpec((B, C_OUT), lambda i: (0, 0)),
        ],
        out_shape=[
            jax.ShapeDtypeStruct((B, 2), jnp.int32),
            jax.ShapeDtypeStruct((B, 2), f32),
            jax.ShapeDtypeStruct((1, 1), f32),
            jax.ShapeDtypeStruct((B, C_OUT), f32),
        ],
        scratch_shapes=[pltpu.VMEM((B, C_IN), f32)],
    )(x3, w_gate, b_exp)

    W2 = W_exp.transpose(0, 3, 4, 1, 2).reshape(E, KS * KS * C_OUT, C_IN)
    bc3 = b_c.reshape(B, C_OUT, 1)
    WB = KS * KS * C_OUT
    BB = 4

    def _w_map(j, k):
        return lambda s, idx, gg: (idx[BB * s + j, k], 0, 0)

    grid_spec = pltpu.PrefetchScalarGridSpec(
        num_scalar_prefetch=2,
        grid=(B // BB,),
        in_specs=[
            pl.BlockSpec((BB, C_IN, HW), lambda s, idx, gg: (s, 0, 0)),
        ] + [
            pl.BlockSpec((1, WB, C_IN), _w_map(j, k))
            for j in range(BB) for k in (0, 1)
        ] + [
            pl.BlockSpec((BB, C_OUT, 1), lambda s, idx, gg: (s, 0, 0)),
        ],
        out_specs=pl.BlockSpec((BB, C_OUT, HW),
                               lambda s, idx, gg: (s, 0, 0)),
    )
    y_flat = pl.pallas_call(
        _make_conv_body(C_IN, C_OUT, H, W, KS),
        grid_spec=grid_spec,
        out_shape=jax.ShapeDtypeStruct((B, C_OUT, HW), f32),
    )(idx2, gg2, x3, *([W2] * (2 * BB)), bc3)
    y = y_flat.reshape(B, C_OUT, H, W)

    return (y, loss_arr[0, 0])
